# baseline (speedup 1.0000x reference)
import functools

import jax
import jax.numpy as jnp
from jax import lax
from jax.experimental import pallas as pl
from jax.experimental.pallas import tpu as pltpu
from jax.experimental.pallas import tpu_sc as plsc

N = 10000
E = 320000
F_IN = 128
H = 128
C = 40
CP = 48

NC = 2
NS = 16
NW = NC * NS
EPW = E // NW
K = 128
NCHUNK = 80
EPWP = NCHUNK * K
NP = 10240
RPT = NP // NS
ZR = 128
GSPLIT = 4


def _make_sc_conv(DIN, DOUT):
    mesh = plsc.VectorSubcoreMesh(core_axis_name="c", subcore_axis_name="s")

    @functools.partial(
        pl.kernel,
        out_type=jax.ShapeDtypeStruct((NC * NP, DOUT), jnp.float32),
        mesh=mesh,
        scratch_types=[
            pltpu.VMEM((3, K), jnp.int32),
            pltpu.VMEM((3, K), jnp.int32),
            pltpu.VMEM((3, K), jnp.int32),
            pltpu.VMEM((3, K), jnp.int32),
            pltpu.VMEM((K, DIN), jnp.bfloat16),
            pltpu.VMEM((K, DIN), jnp.bfloat16),
            pltpu.VMEM((K, DOUT), jnp.float32),
            pltpu.VMEM_SHARED((NP, DOUT), jnp.float32),
            pltpu.SemaphoreType.DMA,
            pltpu.SemaphoreType.DMA,
            pltpu.SemaphoreType.DMA,
            pltpu.SemaphoreType.DMA,
        ],
        compiler_params=pltpu.CompilerParams(use_tc_tiling_on_sc=False,
                                             needs_layout_passes=False),
    )
    def conv(h_hbm, slab_hbm, out_hbm,
             sb0, sb1, sb2, sb3, in0, in1, outb, acc,
             gs0, gs1, ssem, slsem):
        c = lax.axis_index("c")
        s = lax.axis_index("s")
        wid = s * NC + c
        inb = (in0, in1)
        sbuf = (sb0, sb1, sb2, sb3)
        gsem = (gs0, gs1)

        def zrow(i, carry):
            for t in range(DOUT // 16):
                outb[i, pl.ds(16 * t, 16)] = jnp.zeros((16,), jnp.float32)
            return carry

        lax.fori_loop(0, K, zrow, 0)
        base_r = s * RPT
        for rep in range(RPT // K):
            pltpu.sync_copy(outb, acc.at[pl.ds(base_r + rep * K, K)])

        pltpu.sync_copy(slab_hbm.at[wid * NCHUNK + 0], sb0)
        pltpu.sync_copy(slab_hbm.at[wid * NCHUNK + 1], sb1)
        pltpu.async_copy(h_hbm.at[sb0.at[0]], in0, gs0)
        plsc.subcore_barrier()

        def do_chunk(j, q):
            b = q % 2
            nb = 1 - b
            sb = sbuf[q]
            @pl.when(j + 1 < NCHUNK)
            def _():
                @pl.when(j >= 1)
                def _():
                    pltpu.make_async_copy(
                        slab_hbm.at[wid * NCHUNK], sbuf[(q + 1) % 4], slsem).wait()
                pltpu.async_copy(h_hbm.at[sbuf[(q + 1) % 4].at[0]],
                                 inb[nb], gsem[nb])

            @pl.when(j + 2 < NCHUNK)
            def _():
                pltpu.async_copy(slab_hbm.at[wid * NCHUNK + j + 2],
                                 sbuf[(q + 2) % 4], slsem)

            pltpu.make_async_copy(h_hbm.at[sb.at[0]], inb[b], gsem[b]).wait()

            @pl.when(j >= 1)
            def _():
                pltpu.make_async_copy(
                    outb, acc.at[sbuf[(q + 3) % 4].at[1]], ssem).wait()

            def scale(i, cc):
                iv = jnp.full((16,), i, jnp.int32)
                w = plsc.bitcast(
                    plsc.load_gather(sb, [jnp.full((16,), 2, jnp.int32), iv]),
                    jnp.float32)
                for t in range(DIN // 32):
                    v = plsc.bitcast(inb[b][i, pl.ds(32 * t, 32)], jnp.int32)
                    lo = plsc.bitcast(v << 16, jnp.float32)
                    outb[i, pl.ds(32 * t, 16)] = lo * w
                    if 32 * t + 16 < DOUT:
                        hi = plsc.bitcast(v & jnp.int32(-65536), jnp.float32)
                        outb[i, pl.ds(32 * t + 16, 16)] = hi * w
                return cc

            lax.fori_loop(0, K, scale, 0)
            pltpu.async_copy(outb, acc.at[sb.at[1]], ssem, add=True)

        def quad(g, carry):
            for q in range(4):
                do_chunk(4 * g + q, q)
            return carry

        lax.fori_loop(0, NCHUNK // 4, quad, 0)
        pltpu.make_async_copy(outb, acc.at[sbuf[3].at[1]], ssem).wait()
        plsc.subcore_barrier()

        pltpu.sync_copy(acc.at[pl.ds(base_r, RPT)],
                        out_hbm.at[pl.ds(c * NP + base_r, RPT)])

    return conv


_sc_conv_h = _make_sc_conv(H, H)
_sc_conv_c = _make_sc_conv(64, CP)


def _perm(d):
    p = []
    for t in range(d // 32):
        for k in range(16):
            p.append(32 * t + k)
            p.append(32 * t + 16 + k)
    return jnp.array(p, dtype=jnp.int32)


_PERM_H = _perm(H)
_PERM_C = _perm(64)


def _pack_edges(ei, ew):
    pad = EPWP - EPW
    src = jnp.pad(ei[0].reshape(NW, EPW), ((0, 0), (0, pad)))
    dst = jnp.pad(ei[1].reshape(NW, EPW), ((0, 0), (0, pad)))
    w = jnp.pad(ew.reshape(NW, EPW), ((0, 0), (0, pad)))
    wbits = lax.bitcast_convert_type(w, jnp.int32)
    slab = jnp.stack([src.reshape(NW, NCHUNK, K),
                      dst.reshape(NW, NCHUNK, K),
                      wbits.reshape(NW, NCHUNK, K)], axis=2)
    return slab.reshape(NW * NCHUNK, 3, K)


def _t1_body(f_ref, w0_ref, b0_ref, w1_ref, w2_ref, x0_ref, h1_ref, h2_ref):
    f = f_ref[...]
    x0_ref[...] = jnp.dot(f, w0_ref[...], preferred_element_type=jnp.float32) + b0_ref[...]
    h1_ref[...] = jnp.dot(f, w1_ref[...], preferred_element_type=jnp.float32).astype(jnp.bfloat16)
    h2_ref[...] = jnp.dot(f, w2_ref[...], preferred_element_type=jnp.float32).astype(jnp.bfloat16)


def _t1(features, ln1_W, ln1_b, c1_W, c2_W):
    nb = 5
    rb = N // nb
    return pl.pallas_call(
        _t1_body,
        grid=(nb,),
        in_specs=[
            pl.BlockSpec((rb, F_IN), lambda i: (i, 0)),
            pl.BlockSpec((F_IN, H), lambda i: (0, 0)),
            pl.BlockSpec((1, H), lambda i: (0, 0)),
            pl.BlockSpec((F_IN, H), lambda i: (0, 0)),
            pl.BlockSpec((F_IN, H), lambda i: (0, 0)),
        ],
        out_specs=[
            pl.BlockSpec((rb, H), lambda i: (i, 0)),
            pl.BlockSpec((rb, H), lambda i: (i, 0)),
            pl.BlockSpec((rb, H), lambda i: (i, 0)),
        ],
        out_shape=[jax.ShapeDtypeStruct((N, H), jnp.float32),
                   jax.ShapeDtypeStruct((N, H), jnp.bfloat16),
                   jax.ShapeDtypeStruct((N, H), jnp.bfloat16)],
    )(features, ln1_W, ln1_b.reshape(1, H), c1_W, c2_W)


def _t2_body(x0_ref, p1_ref, p2_ref, c1b_ref, c2b_ref, g_ref, b_ref,
             w0_ref, b0_ref, w3_ref, w4_ref, y0_ref, g3_ref, g4_ref):
    x0 = x0_ref[...]
    x1 = p1_ref[0:N, :] + p1_ref[NP:NP + N, :] + c1b_ref[...]
    x2 = p2_ref[0:N, :] + p2_ref[NP:NP + N, :] + c2b_ref[...]
    h = jnp.concatenate([x0, x1, x2], axis=1)
    m = jnp.mean(h, axis=0, keepdims=True)
    v = jnp.mean(h * h, axis=0, keepdims=True) - m * m
    hb = g_ref[...] * (h - m) * lax.rsqrt(v + 1e-5) + b_ref[...]
    y0_ref[...] = jnp.dot(hb, w0_ref[...], preferred_element_type=jnp.float32) + b0_ref[...]
    g3_ref[...] = jnp.dot(hb, w3_ref[...], preferred_element_type=jnp.float32).astype(jnp.bfloat16)
    g4_ref[...] = jnp.dot(hb, w4_ref[...], preferred_element_type=jnp.float32).astype(jnp.bfloat16)


def _t2(x0, p1, p2, c1_b, c2_b, bn1_g, bn1_b, ln2_W, ln2_b, c3_Wp, c4_Wp):
    return pl.pallas_call(
        _t2_body,
        out_shape=[
            jax.ShapeDtypeStruct((N, C), jnp.float32),
            jax.ShapeDtypeStruct((N, 64), jnp.bfloat16),
            jax.ShapeDtypeStruct((N, 64), jnp.bfloat16),
        ],
        compiler_params=pltpu.CompilerParams(vmem_limit_bytes=100 * 1024 * 1024),
    )(x0, p1, p2, c1_b.reshape(1, H), c2_b.reshape(1, H),
      bn1_g.reshape(1, 3 * H), bn1_b.reshape(1, 3 * H),
      ln2_W, ln2_b.reshape(1, C), c3_Wp, c4_Wp)


def _pad_perm_c(w):
    return jnp.take(jnp.pad(w, ((0, 0), (0, 64 - C))), _PERM_C, axis=1)


def _t3_body(y0_ref, q1_ref, q2_ref, c3b_ref, c4b_ref, g_ref, b_ref,
             w_ref, wb_ref, out_ref):
    y0 = y0_ref[...]
    y1 = q1_ref[0:N, 0:C] + q1_ref[NP:NP + N, 0:C] + c3b_ref[...]
    y2 = q2_ref[0:N, 0:C] + q2_ref[NP:NP + N, 0:C] + c4b_ref[...]
    z = jnp.concatenate([y0, y1, y2], axis=1)
    m = jnp.mean(z, axis=0, keepdims=True)
    v = jnp.mean(z * z, axis=0, keepdims=True) - m * m
    zb = g_ref[...] * (z - m) * lax.rsqrt(v + 1e-5) + b_ref[...]
    out_ref[...] = jnp.dot(zb, w_ref[...], preferred_element_type=jnp.float32) + wb_ref[...]


def _t3(y0, q1, q2, c3_b, c4_b, bn2_g, bn2_b, conv_W, conv_b):
    return pl.pallas_call(
        _t3_body,
        out_shape=jax.ShapeDtypeStruct((N, C), jnp.float32),
    )(y0, q1, q2, c3_b.reshape(1, C), c4_b.reshape(1, C),
      bn2_g.reshape(1, 3 * C), bn2_b.reshape(1, 3 * C),
      conv_W, conv_b.reshape(1, C))


def kernel(features, edge_index, edge_index2, edge_weight, edge_weight2,
           ln1_W, ln1_b, c1_W, c1_b, c2_W, c2_b, bn1_g, bn1_b,
           ln2_W, ln2_b, c3_W, c3_b, c4_W, c4_b, bn2_g, bn2_b,
           conv_W, conv_b):
    slab1 = _pack_edges(edge_index, edge_weight)
    slab2 = _pack_edges(edge_index2, edge_weight2)

    x0, h1, h2 = _t1(features, ln1_W, ln1_b,
                     jnp.take(c1_W, _PERM_H, axis=1),
                     jnp.take(c2_W, _PERM_H, axis=1))
    p1 = _sc_conv_h(h1, slab1)
    p2 = _sc_conv_h(h2, slab2)

    y0, g3, g4 = _t2(x0, p1, p2, c1_b, c2_b, bn1_g, bn1_b,
                     ln2_W, ln2_b, _pad_perm_c(c3_W), _pad_perm_c(c4_W))

    q1 = _sc_conv_c(g3, slab1)
    q2 = _sc_conv_c(g4, slab2)

    return _t3(y0, q1, q2, c3_b, c4_b, bn2_g, bn2_b, conv_W, conv_b)


# speedup vs baseline: 1.1046x; 1.1046x over previous
import functools

import jax
import jax.numpy as jnp
from jax import lax
from jax.experimental import pallas as pl
from jax.experimental.pallas import tpu as pltpu
from jax.experimental.pallas import tpu_sc as plsc

N = 10000
E = 320000
F_IN = 128
H = 128
C = 40
CP = 48

NC = 2
NS = 16
NW = NC * NS
EPW = E // NW
K = 128
NCHUNK = 80
EPWP = NCHUNK * K
NP = N
SPT = 632


def _make_sc_conv(DIN, DOUT):
    mesh = plsc.VectorSubcoreMesh(core_axis_name="c", subcore_axis_name="s")

    @functools.partial(
        pl.kernel,
        out_type=jax.ShapeDtypeStruct((NC * NP, DOUT), jnp.float32),
        mesh=mesh,
        scratch_types=[
            pltpu.VMEM((3, K), jnp.int32),
            pltpu.VMEM((3, K), jnp.int32),
            pltpu.VMEM((3, K), jnp.int32),
            pltpu.VMEM((3, K), jnp.int32),
            pltpu.VMEM((K, DIN), jnp.bfloat16),
            pltpu.VMEM((K, DIN), jnp.bfloat16),
            pltpu.VMEM((K, DOUT), jnp.float32),
            pltpu.VMEM((K, DOUT), jnp.float32),
            pltpu.VMEM_SHARED((NP, DOUT), jnp.float32),
            pltpu.SemaphoreType.DMA,
            pltpu.SemaphoreType.DMA,
            pltpu.SemaphoreType.DMA,
            pltpu.SemaphoreType.DMA,
            pltpu.SemaphoreType.DMA,
        ],
        compiler_params=pltpu.CompilerParams(use_tc_tiling_on_sc=False,
                                             needs_layout_passes=False),
    )
    def conv(h_hbm, slab_hbm, out_hbm,
             sb0, sb1, sb2, sb3, in0, in1, out0, out1, acc,
             gs0, gs1, ss0, ss1, slsem):
        c = lax.axis_index("c")
        s = lax.axis_index("s")
        wid = s * NC + c
        inb = (in0, in1)
        outb = (out0, out1)
        sbuf = (sb0, sb1, sb2, sb3)
        gsem = (gs0, gs1)
        ssem = (ss0, ss1)

        def zrow(i, carry):
            for t in range(DOUT // 16):
                out0[i, pl.ds(16 * t, 16)] = jnp.zeros((16,), jnp.float32)
            return carry

        lax.fori_loop(0, K, zrow, 0)
        base_r = s * SPT

        @pl.when(s < NS - 1)
        def _():
            for rep in range(4):
                pltpu.sync_copy(out0, acc.at[pl.ds(base_r + rep * K, K)])
            pltpu.sync_copy(out0.at[pl.ds(0, SPT - 4 * K)],
                            acc.at[pl.ds(base_r + 4 * K, SPT - 4 * K)])

        @pl.when(s == NS - 1)
        def _():
            for rep in range(4):
                pltpu.sync_copy(out0, acc.at[pl.ds(base_r + rep * K, K)])
            pltpu.sync_copy(out0.at[pl.ds(0, NP - 15 * SPT - 4 * K)],
                            acc.at[pl.ds(base_r + 4 * K, NP - 15 * SPT - 4 * K)])

        pltpu.sync_copy(slab_hbm.at[wid * NCHUNK + 0], sb0)
        pltpu.sync_copy(slab_hbm.at[wid * NCHUNK + 1], sb1)
        pltpu.async_copy(h_hbm.at[sb0.at[0]], in0, gs0)
        plsc.subcore_barrier()

        def do_chunk(j, q):
            b = q % 2
            nb = 1 - b
            sb = sbuf[q]
            @pl.when(j >= 2)
            def _():
                pltpu.make_async_copy(
                    outb[b], acc.at[sbuf[(q + 2) % 4].at[1]], ssem[b]).wait()

            @pl.when(j + 1 < NCHUNK)
            def _():
                @pl.when(j >= 1)
                def _():
                    pltpu.make_async_copy(
                        slab_hbm.at[wid * NCHUNK], sbuf[(q + 1) % 4], slsem).wait()
                pltpu.async_copy(h_hbm.at[sbuf[(q + 1) % 4].at[0]],
                                 inb[nb], gsem[nb])

            @pl.when(j + 2 < NCHUNK)
            def _():
                pltpu.async_copy(slab_hbm.at[wid * NCHUNK + j + 2],
                                 sbuf[(q + 2) % 4], slsem)

            pltpu.make_async_copy(h_hbm.at[sb.at[0]], inb[b], gsem[b]).wait()

            def scale(i, cc):
                iv = jnp.full((16,), i, jnp.int32)
                w = plsc.bitcast(
                    plsc.load_gather(sb, [jnp.full((16,), 2, jnp.int32), iv]),
                    jnp.float32)
                for t in range(DIN // 32):
                    v = plsc.bitcast(inb[b][i, pl.ds(32 * t, 32)], jnp.int32)
                    lo = plsc.bitcast(v << 16, jnp.float32)
                    outb[b][i, pl.ds(32 * t, 16)] = lo * w
                    if 32 * t + 16 < DOUT:
                        hi = plsc.bitcast(v & jnp.int32(-65536), jnp.float32)
                        outb[b][i, pl.ds(32 * t + 16, 16)] = hi * w
                return cc

            lax.fori_loop(0, K, scale, 0)
            pltpu.async_copy(outb[b], acc.at[sb.at[1]], ssem[b], add=True)

        def quad(g, carry):
            for q in range(4):
                do_chunk(4 * g + q, q)
            return carry

        lax.fori_loop(0, NCHUNK // 4, quad, 0)
        pltpu.make_async_copy(outb[0], acc.at[sbuf[2].at[1]], ssem[0]).wait()
        pltpu.make_async_copy(outb[1], acc.at[sbuf[3].at[1]], ssem[1]).wait()
        plsc.subcore_barrier()

        @pl.when(s < NS - 1)
        def _():
            pltpu.sync_copy(acc.at[pl.ds(base_r, SPT)],
                            out_hbm.at[pl.ds(c * NP + base_r, SPT)])

        @pl.when(s == NS - 1)
        def _():
            pltpu.sync_copy(acc.at[pl.ds(base_r, NP - 15 * SPT)],
                            out_hbm.at[pl.ds(c * NP + base_r, NP - 15 * SPT)])

    return conv


_sc_conv_h = _make_sc_conv(H, H)
_sc_conv_c = _make_sc_conv(64, CP)


def _perm(d):
    p = []
    for t in range(d // 32):
        for k in range(16):
            p.append(32 * t + k)
            p.append(32 * t + 16 + k)
    return jnp.array(p, dtype=jnp.int32)


_PERM_H = _perm(H)
_PERM_C = _perm(64)


def _pack_edges(ei, ew):
    pad = EPWP - EPW
    src = jnp.pad(ei[0].reshape(NW, EPW), ((0, 0), (0, pad)))
    dst = jnp.pad(ei[1].reshape(NW, EPW), ((0, 0), (0, pad)))
    w = jnp.pad(ew.reshape(NW, EPW), ((0, 0), (0, pad)))
    wbits = lax.bitcast_convert_type(w, jnp.int32)
    slab = jnp.stack([src.reshape(NW, NCHUNK, K),
                      dst.reshape(NW, NCHUNK, K),
                      wbits.reshape(NW, NCHUNK, K)], axis=2)
    return slab.reshape(NW * NCHUNK, 3, K)


def _t1_body(f_ref, w0_ref, b0_ref, w1_ref, w2_ref, x0_ref, h1_ref, h2_ref):
    f = f_ref[...]
    x0_ref[...] = jnp.dot(f, w0_ref[...], preferred_element_type=jnp.float32) + b0_ref[...]
    h1_ref[...] = jnp.dot(f, w1_ref[...], preferred_element_type=jnp.float32).astype(jnp.bfloat16)
    h2_ref[...] = jnp.dot(f, w2_ref[...], preferred_element_type=jnp.float32).astype(jnp.bfloat16)


def _t1(features, ln1_W, ln1_b, c1_W, c2_W):
    nb = 5
    rb = N // nb
    return pl.pallas_call(
        _t1_body,
        grid=(nb,),
        in_specs=[
            pl.BlockSpec((rb, F_IN), lambda i: (i, 0)),
            pl.BlockSpec((F_IN, H), lambda i: (0, 0)),
            pl.BlockSpec((1, H), lambda i: (0, 0)),
            pl.BlockSpec((F_IN, H), lambda i: (0, 0)),
            pl.BlockSpec((F_IN, H), lambda i: (0, 0)),
        ],
        out_specs=[
            pl.BlockSpec((rb, H), lambda i: (i, 0)),
            pl.BlockSpec((rb, H), lambda i: (i, 0)),
            pl.BlockSpec((rb, H), lambda i: (i, 0)),
        ],
        out_shape=[jax.ShapeDtypeStruct((N, H), jnp.float32),
                   jax.ShapeDtypeStruct((N, H), jnp.bfloat16),
                   jax.ShapeDtypeStruct((N, H), jnp.bfloat16)],
    )(features, ln1_W, ln1_b.reshape(1, H), c1_W, c2_W)


def _t2_body(x0_ref, p1_ref, p2_ref, c1b_ref, c2b_ref, g_ref, b_ref,
             w0_ref, b0_ref, w3_ref, w4_ref, y0_ref, g3_ref, g4_ref):
    x0 = x0_ref[...]
    x1 = p1_ref[0:N, :] + p1_ref[NP:NP + N, :] + c1b_ref[...]
    x2 = p2_ref[0:N, :] + p2_ref[NP:NP + N, :] + c2b_ref[...]
    h = jnp.concatenate([x0, x1, x2], axis=1)
    m = jnp.mean(h, axis=0, keepdims=True)
    v = jnp.mean(h * h, axis=0, keepdims=True) - m * m
    hb = g_ref[...] * (h - m) * lax.rsqrt(v + 1e-5) + b_ref[...]
    y0_ref[...] = jnp.dot(hb, w0_ref[...], preferred_element_type=jnp.float32) + b0_ref[...]
    g3_ref[...] = jnp.dot(hb, w3_ref[...], preferred_element_type=jnp.float32).astype(jnp.bfloat16)
    g4_ref[...] = jnp.dot(hb, w4_ref[...], preferred_element_type=jnp.float32).astype(jnp.bfloat16)


def _t2(x0, p1, p2, c1_b, c2_b, bn1_g, bn1_b, ln2_W, ln2_b, c3_Wp, c4_Wp):
    return pl.pallas_call(
        _t2_body,
        out_shape=[
            jax.ShapeDtypeStruct((N, C), jnp.float32),
            jax.ShapeDtypeStruct((N, 64), jnp.bfloat16),
            jax.ShapeDtypeStruct((N, 64), jnp.bfloat16),
        ],
        compiler_params=pltpu.CompilerParams(vmem_limit_bytes=100 * 1024 * 1024),
    )(x0, p1, p2, c1_b.reshape(1, H), c2_b.reshape(1, H),
      bn1_g.reshape(1, 3 * H), bn1_b.reshape(1, 3 * H),
      ln2_W, ln2_b.reshape(1, C), c3_Wp, c4_Wp)


def _pad_perm_c(w):
    return jnp.take(jnp.pad(w, ((0, 0), (0, 64 - C))), _PERM_C, axis=1)


def _t3_body(y0_ref, q1_ref, q2_ref, c3b_ref, c4b_ref, g_ref, b_ref,
             w_ref, wb_ref, out_ref):
    y0 = y0_ref[...]
    y1 = q1_ref[0:N, 0:C] + q1_ref[NP:NP + N, 0:C] + c3b_ref[...]
    y2 = q2_ref[0:N, 0:C] + q2_ref[NP:NP + N, 0:C] + c4b_ref[...]
    z = jnp.concatenate([y0, y1, y2], axis=1)
    m = jnp.mean(z, axis=0, keepdims=True)
    v = jnp.mean(z * z, axis=0, keepdims=True) - m * m
    zb = g_ref[...] * (z - m) * lax.rsqrt(v + 1e-5) + b_ref[...]
    out_ref[...] = jnp.dot(zb, w_ref[...], preferred_element_type=jnp.float32) + wb_ref[...]


def _t3(y0, q1, q2, c3_b, c4_b, bn2_g, bn2_b, conv_W, conv_b):
    return pl.pallas_call(
        _t3_body,
        out_shape=jax.ShapeDtypeStruct((N, C), jnp.float32),
    )(y0, q1, q2, c3_b.reshape(1, C), c4_b.reshape(1, C),
      bn2_g.reshape(1, 3 * C), bn2_b.reshape(1, 3 * C),
      conv_W, conv_b.reshape(1, C))


def kernel(features, edge_index, edge_index2, edge_weight, edge_weight2,
           ln1_W, ln1_b, c1_W, c1_b, c2_W, c2_b, bn1_g, bn1_b,
           ln2_W, ln2_b, c3_W, c3_b, c4_W, c4_b, bn2_g, bn2_b,
           conv_W, conv_b):
    slab1 = _pack_edges(edge_index, edge_weight)
    slab2 = _pack_edges(edge_index2, edge_weight2)

    x0, h1, h2 = _t1(features, ln1_W, ln1_b,
                     jnp.take(c1_W, _PERM_H, axis=1),
                     jnp.take(c2_W, _PERM_H, axis=1))
    p1 = _sc_conv_h(h1, slab1)
    p2 = _sc_conv_h(h2, slab2)

    y0, g3, g4 = _t2(x0, p1, p2, c1_b, c2_b, bn1_g, bn1_b,
                     ln2_W, ln2_b, _pad_perm_c(c3_W), _pad_perm_c(c4_W))

    q1 = _sc_conv_c(g3, slab1)
    q2 = _sc_conv_c(g4, slab2)

    return _t3(y0, q1, q2, c3_b, c4_b, bn2_g, bn2_b, conv_W, conv_b)


# speedup vs baseline: 1.1047x; 1.0001x over previous
import functools

import jax
import jax.numpy as jnp
from jax import lax
from jax.experimental import pallas as pl
from jax.experimental.pallas import tpu as pltpu
from jax.experimental.pallas import tpu_sc as plsc

N = 10000
E = 320000
F_IN = 128
H = 128
C = 40
CP = 48

NC = 2
NS = 16
NW = NC * NS
EPW = E // NW
K = 128
NCHUNK = 80
EPWP = NCHUNK * K
NP = N
SPT = 632


def _make_sc_conv(DIN, DOUT):
    mesh = plsc.VectorSubcoreMesh(core_axis_name="c", subcore_axis_name="s")

    @functools.partial(
        pl.kernel,
        out_type=jax.ShapeDtypeStruct((NC * NP, DOUT), jnp.float32),
        mesh=mesh,
        scratch_types=[
            pltpu.VMEM((3, K), jnp.int32),
            pltpu.VMEM((3, K), jnp.int32),
            pltpu.VMEM((3, K), jnp.int32),
            pltpu.VMEM((3, K), jnp.int32),
            pltpu.VMEM((K, DIN), jnp.bfloat16),
            pltpu.VMEM((K, DIN), jnp.bfloat16),
            pltpu.VMEM((K, DOUT), jnp.float32),
            pltpu.VMEM((K, DOUT), jnp.float32),
            pltpu.VMEM_SHARED((NP, DOUT), jnp.float32),
            pltpu.SemaphoreType.DMA,
            pltpu.SemaphoreType.DMA,
            pltpu.SemaphoreType.DMA,
            pltpu.SemaphoreType.DMA,
            pltpu.SemaphoreType.DMA,
        ],
        compiler_params=pltpu.CompilerParams(use_tc_tiling_on_sc=False,
                                             needs_layout_passes=False),
    )
    def conv(h_hbm, slab_hbm, out_hbm,
             sb0, sb1, sb2, sb3, in0, in1, out0, out1, acc,
             gs0, gs1, ss0, ss1, slsem):
        c = lax.axis_index("c")
        s = lax.axis_index("s")
        wid = s * NC + c
        inb = (in0, in1)
        outb = (out0, out1)
        sbuf = (sb0, sb1, sb2, sb3)
        gsem = (gs0, gs1)
        ssem = (ss0, ss1)

        def zrow(i, carry):
            for t in range(DOUT // 16):
                out0[i, pl.ds(16 * t, 16)] = jnp.zeros((16,), jnp.float32)
            return carry

        lax.fori_loop(0, K, zrow, 0)
        base_r = s * SPT

        @pl.when(s < NS - 1)
        def _():
            for rep in range(4):
                pltpu.sync_copy(out0, acc.at[pl.ds(base_r + rep * K, K)])
            pltpu.sync_copy(out0.at[pl.ds(0, SPT - 4 * K)],
                            acc.at[pl.ds(base_r + 4 * K, SPT - 4 * K)])

        @pl.when(s == NS - 1)
        def _():
            for rep in range(4):
                pltpu.sync_copy(out0, acc.at[pl.ds(base_r + rep * K, K)])
            pltpu.sync_copy(out0.at[pl.ds(0, NP - 15 * SPT - 4 * K)],
                            acc.at[pl.ds(base_r + 4 * K, NP - 15 * SPT - 4 * K)])

        pltpu.sync_copy(slab_hbm.at[wid * NCHUNK + 0], sb0)
        pltpu.sync_copy(slab_hbm.at[wid * NCHUNK + 1], sb1)
        pltpu.async_copy(h_hbm.at[sb0.at[0]], in0, gs0)
        plsc.subcore_barrier()

        def do_chunk(j, q):
            b = q % 2
            nb = 1 - b
            sb = sbuf[q]
            @pl.when(j >= 2)
            def _():
                pltpu.make_async_copy(
                    outb[b], acc.at[sbuf[(q + 2) % 4].at[1]], ssem[b]).wait()

            @pl.when(j + 1 < NCHUNK)
            def _():
                @pl.when(j >= 1)
                def _():
                    pltpu.make_async_copy(
                        slab_hbm.at[wid * NCHUNK], sbuf[(q + 1) % 4], slsem).wait()
                pltpu.async_copy(h_hbm.at[sbuf[(q + 1) % 4].at[0]],
                                 inb[nb], gsem[nb])

            @pl.when(j + 2 < NCHUNK)
            def _():
                pltpu.async_copy(slab_hbm.at[wid * NCHUNK + j + 2],
                                 sbuf[(q + 2) % 4], slsem)

            pltpu.make_async_copy(h_hbm.at[sb.at[0]], inb[b], gsem[b]).wait()

            def scale(i2, cc):
                for u in range(2):
                    i = 2 * i2 + u
                    iv = jnp.full((16,), i, jnp.int32)
                    w = plsc.bitcast(
                        plsc.load_gather(sb, [jnp.full((16,), 2, jnp.int32), iv]),
                        jnp.float32)
                    for t in range(DIN // 32):
                        v = plsc.bitcast(inb[b][i, pl.ds(32 * t, 32)], jnp.int32)
                        lo = plsc.bitcast(v << 16, jnp.float32)
                        outb[b][i, pl.ds(32 * t, 16)] = lo * w
                        if 32 * t + 16 < DOUT:
                            hi = plsc.bitcast(v & jnp.int32(-65536), jnp.float32)
                            outb[b][i, pl.ds(32 * t + 16, 16)] = hi * w
                return cc

            lax.fori_loop(0, K // 2, scale, 0)
            pltpu.async_copy(outb[b], acc.at[sb.at[1]], ssem[b], add=True)

        def quad(g, carry):
            for q in range(4):
                do_chunk(4 * g + q, q)
            return carry

        lax.fori_loop(0, NCHUNK // 4, quad, 0)
        pltpu.make_async_copy(outb[0], acc.at[sbuf[2].at[1]], ssem[0]).wait()
        pltpu.make_async_copy(outb[1], acc.at[sbuf[3].at[1]], ssem[1]).wait()
        plsc.subcore_barrier()

        @pl.when(s < NS - 1)
        def _():
            pltpu.sync_copy(acc.at[pl.ds(base_r, SPT)],
                            out_hbm.at[pl.ds(c * NP + base_r, SPT)])

        @pl.when(s == NS - 1)
        def _():
            pltpu.sync_copy(acc.at[pl.ds(base_r, NP - 15 * SPT)],
                            out_hbm.at[pl.ds(c * NP + base_r, NP - 15 * SPT)])

    return conv


_sc_conv_h = _make_sc_conv(H, H)
_sc_conv_c = _make_sc_conv(64, CP)


def _perm(d):
    p = []
    for t in range(d // 32):
        for k in range(16):
            p.append(32 * t + k)
            p.append(32 * t + 16 + k)
    return jnp.array(p, dtype=jnp.int32)


_PERM_H = _perm(H)
_PERM_C = _perm(64)


def _pack_edges(ei, ew):
    pad = EPWP - EPW
    src = jnp.pad(ei[0].reshape(NW, EPW), ((0, 0), (0, pad)))
    dst = jnp.pad(ei[1].reshape(NW, EPW), ((0, 0), (0, pad)))
    w = jnp.pad(ew.reshape(NW, EPW), ((0, 0), (0, pad)))
    wbits = lax.bitcast_convert_type(w, jnp.int32)
    slab = jnp.stack([src.reshape(NW, NCHUNK, K),
                      dst.reshape(NW, NCHUNK, K),
                      wbits.reshape(NW, NCHUNK, K)], axis=2)
    return slab.reshape(NW * NCHUNK, 3, K)


def _t1_body(f_ref, w0_ref, b0_ref, w1_ref, w2_ref, x0_ref, h1_ref, h2_ref):
    f = f_ref[...]
    x0_ref[...] = jnp.dot(f, w0_ref[...], preferred_element_type=jnp.float32) + b0_ref[...]
    h1_ref[...] = jnp.dot(f, w1_ref[...], preferred_element_type=jnp.float32).astype(jnp.bfloat16)
    h2_ref[...] = jnp.dot(f, w2_ref[...], preferred_element_type=jnp.float32).astype(jnp.bfloat16)


def _t1(features, ln1_W, ln1_b, c1_W, c2_W):
    nb = 5
    rb = N // nb
    return pl.pallas_call(
        _t1_body,
        grid=(nb,),
        in_specs=[
            pl.BlockSpec((rb, F_IN), lambda i: (i, 0)),
            pl.BlockSpec((F_IN, H), lambda i: (0, 0)),
            pl.BlockSpec((1, H), lambda i: (0, 0)),
            pl.BlockSpec((F_IN, H), lambda i: (0, 0)),
            pl.BlockSpec((F_IN, H), lambda i: (0, 0)),
        ],
        out_specs=[
            pl.BlockSpec((rb, H), lambda i: (i, 0)),
            pl.BlockSpec((rb, H), lambda i: (i, 0)),
            pl.BlockSpec((rb, H), lambda i: (i, 0)),
        ],
        out_shape=[jax.ShapeDtypeStruct((N, H), jnp.float32),
                   jax.ShapeDtypeStruct((N, H), jnp.bfloat16),
                   jax.ShapeDtypeStruct((N, H), jnp.bfloat16)],
    )(features, ln1_W, ln1_b.reshape(1, H), c1_W, c2_W)


def _t2_body(x0_ref, p1_ref, p2_ref, c1b_ref, c2b_ref, g_ref, b_ref,
             w0_ref, b0_ref, w3_ref, w4_ref, y0_ref, g3_ref, g4_ref):
    x0 = x0_ref[...]
    x1 = p1_ref[0:N, :] + p1_ref[NP:NP + N, :] + c1b_ref[...]
    x2 = p2_ref[0:N, :] + p2_ref[NP:NP + N, :] + c2b_ref[...]
    h = jnp.concatenate([x0, x1, x2], axis=1)
    m = jnp.mean(h, axis=0, keepdims=True)
    v = jnp.mean(h * h, axis=0, keepdims=True) - m * m
    hb = g_ref[...] * (h - m) * lax.rsqrt(v + 1e-5) + b_ref[...]
    y0_ref[...] = jnp.dot(hb, w0_ref[...], preferred_element_type=jnp.float32) + b0_ref[...]
    g3_ref[...] = jnp.dot(hb, w3_ref[...], preferred_element_type=jnp.float32).astype(jnp.bfloat16)
    g4_ref[...] = jnp.dot(hb, w4_ref[...], preferred_element_type=jnp.float32).astype(jnp.bfloat16)


def _t2(x0, p1, p2, c1_b, c2_b, bn1_g, bn1_b, ln2_W, ln2_b, c3_Wp, c4_Wp):
    return pl.pallas_call(
        _t2_body,
        out_shape=[
            jax.ShapeDtypeStruct((N, C), jnp.float32),
            jax.ShapeDtypeStruct((N, 64), jnp.bfloat16),
            jax.ShapeDtypeStruct((N, 64), jnp.bfloat16),
        ],
        compiler_params=pltpu.CompilerParams(vmem_limit_bytes=100 * 1024 * 1024),
    )(x0, p1, p2, c1_b.reshape(1, H), c2_b.reshape(1, H),
      bn1_g.reshape(1, 3 * H), bn1_b.reshape(1, 3 * H),
      ln2_W, ln2_b.reshape(1, C), c3_Wp, c4_Wp)


def _pad_perm_c(w):
    return jnp.take(jnp.pad(w, ((0, 0), (0, 64 - C))), _PERM_C, axis=1)


def _t3_body(y0_ref, q1_ref, q2_ref, c3b_ref, c4b_ref, g_ref, b_ref,
             w_ref, wb_ref, out_ref):
    y0 = y0_ref[...]
    y1 = q1_ref[0:N, 0:C] + q1_ref[NP:NP + N, 0:C] + c3b_ref[...]
    y2 = q2_ref[0:N, 0:C] + q2_ref[NP:NP + N, 0:C] + c4b_ref[...]
    z = jnp.concatenate([y0, y1, y2], axis=1)
    m = jnp.mean(z, axis=0, keepdims=True)
    v = jnp.mean(z * z, axis=0, keepdims=True) - m * m
    zb = g_ref[...] * (z - m) * lax.rsqrt(v + 1e-5) + b_ref[...]
    out_ref[...] = jnp.dot(zb, w_ref[...], preferred_element_type=jnp.float32) + wb_ref[...]


def _t3(y0, q1, q2, c3_b, c4_b, bn2_g, bn2_b, conv_W, conv_b):
    return pl.pallas_call(
        _t3_body,
        out_shape=jax.ShapeDtypeStruct((N, C), jnp.float32),
    )(y0, q1, q2, c3_b.reshape(1, C), c4_b.reshape(1, C),
      bn2_g.reshape(1, 3 * C), bn2_b.reshape(1, 3 * C),
      conv_W, conv_b.reshape(1, C))


def kernel(features, edge_index, edge_index2, edge_weight, edge_weight2,
           ln1_W, ln1_b, c1_W, c1_b, c2_W, c2_b, bn1_g, bn1_b,
           ln2_W, ln2_b, c3_W, c3_b, c4_W, c4_b, bn2_g, bn2_b,
           conv_W, conv_b):
    slab1 = _pack_edges(edge_index, edge_weight)
    slab2 = _pack_edges(edge_index2, edge_weight2)

    x0, h1, h2 = _t1(features, ln1_W, ln1_b,
                     jnp.take(c1_W, _PERM_H, axis=1),
                     jnp.take(c2_W, _PERM_H, axis=1))
    p1 = _sc_conv_h(h1, slab1)
    p2 = _sc_conv_h(h2, slab2)

    y0, g3, g4 = _t2(x0, p1, p2, c1_b, c2_b, bn1_g, bn1_b,
                     ln2_W, ln2_b, _pad_perm_c(c3_W), _pad_perm_c(c4_W))

    q1 = _sc_conv_c(g3, slab1)
    q2 = _sc_conv_c(g4, slab2)

    return _t3(y0, q1, q2, c3_b, c4_b, bn2_g, bn2_b, conv_W, conv_b)
